# Initial kernel scaffold; baseline (speedup 1.0000x reference)
import jax
import jax.numpy as jnp
from jax.experimental import pallas as pl

N_TOKENS = 8192
MODEL_DIM = 4096
H1 = 1024
H2 = 256
NUM_EXPERTS = 64
TOP_K = 8

BT = 256

_PREC = jax.lax.Precision.HIGHEST


def _layernorm(h, gamma, beta, eps=1e-5):
    mu = jnp.mean(h, axis=-1, keepdims=True)
    var = jnp.mean((h - mu) ** 2, axis=-1, keepdims=True)
    return (h - mu) * jax.lax.rsqrt(var + eps) * gamma + beta


def _gelu_exact(h):
    return 0.5 * h * (1.0 + jax.lax.erf(h * (2.0 ** -0.5)))


def _gate_body(x_ref, w1_ref, b1_ref, g1_ref, beta1_ref, w2_ref, b2_ref,
               g2_ref, beta2_ref, w3_ref, w_out_ref, i_out_ref, l_out_ref):
    h = jnp.dot(x_ref[...], w1_ref[...], preferred_element_type=jnp.float32,
                precision=_PREC) + b1_ref[...]
    h = _layernorm(h, g1_ref[...], beta1_ref[...])
    h = _gelu_exact(h)
    h = jnp.dot(h, w2_ref[...], preferred_element_type=jnp.float32,
                precision=_PREC) + b2_ref[...]
    h = _layernorm(h, g2_ref[...], beta2_ref[...])
    h = _gelu_exact(h)
    logits = jnp.dot(h, w3_ref[...], preferred_element_type=jnp.float32,
                     precision=_PREC)
    l_out_ref[...] = logits

    lane = jax.lax.broadcasted_iota(jnp.int32, (BT, NUM_EXPERTS), 1)
    work = logits
    vals, idxs = [], []
    for _ in range(TOP_K):
        m = jnp.max(work, axis=-1, keepdims=True)
        is_m = work == m
        idx = jnp.min(jnp.where(is_m, lane, NUM_EXPERTS), axis=-1,
                      keepdims=True)
        vals.append(m)
        idxs.append(idx)
        work = jnp.where(lane == idx, -jnp.inf, work)
    v = jnp.concatenate(vals, axis=-1)
    i = jnp.concatenate(idxs, axis=-1)
    e = jnp.exp(v - v[:, 0:1])
    w_out_ref[...] = e / jnp.sum(e, axis=-1, keepdims=True)
    i_out_ref[...] = i


@jax.jit
def kernel(x, W1, b1, g1, beta1, W2, b2, g2, beta2, W3):
    grid = (N_TOKENS // BT,)
    full = lambda shape: pl.BlockSpec(shape, lambda i: (0,) * len(shape))
    out_shapes = (
        jax.ShapeDtypeStruct((N_TOKENS, TOP_K), jnp.float32),
        jax.ShapeDtypeStruct((N_TOKENS, TOP_K), jnp.int32),
        jax.ShapeDtypeStruct((N_TOKENS, NUM_EXPERTS), jnp.float32),
    )
    return pl.pallas_call(
        _gate_body,
        grid=grid,
        in_specs=[
            pl.BlockSpec((BT, MODEL_DIM), lambda i: (i, 0)),
            full((MODEL_DIM, H1)),
            full((H1,)),
            full((H1,)),
            full((H1,)),
            full((H1, H2)),
            full((H2,)),
            full((H2,)),
            full((H2,)),
            full((H2, NUM_EXPERTS)),
        ],
        out_specs=(
            pl.BlockSpec((BT, TOP_K), lambda i: (i, 0)),
            pl.BlockSpec((BT, TOP_K), lambda i: (i, 0)),
            pl.BlockSpec((BT, NUM_EXPERTS), lambda i: (i, 0)),
        ),
        out_shape=out_shapes,
    )(x, W1, b1, g1, beta1, W2, b2, g2, beta2, W3)


# speedup vs baseline: 1.3815x; 1.3815x over previous
import jax
import jax.numpy as jnp
from jax.experimental import pallas as pl

N_TOKENS = 8192
MODEL_DIM = 4096
H1 = 1024
H2 = 256
NUM_EXPERTS = 64
TOP_K = 8

BT = 256

_PREC = jax.lax.Precision.DEFAULT


def _layernorm(h, gamma, beta, eps=1e-5):
    mu = jnp.mean(h, axis=-1, keepdims=True)
    var = jnp.mean((h - mu) ** 2, axis=-1, keepdims=True)
    return (h - mu) * jax.lax.rsqrt(var + eps) * gamma + beta


def _gelu_exact(h):
    return 0.5 * h * (1.0 + jax.lax.erf(h * (2.0 ** -0.5)))


def _gate_body(x_ref, w1_ref, b1_ref, g1_ref, beta1_ref, w2_ref, b2_ref,
               g2_ref, beta2_ref, w3_ref, w_out_ref, i_out_ref, l_out_ref):
    h = jnp.dot(x_ref[...], w1_ref[...], preferred_element_type=jnp.float32,
                precision=_PREC) + b1_ref[...]
    h = _layernorm(h, g1_ref[...], beta1_ref[...])
    h = _gelu_exact(h)
    h = jnp.dot(h, w2_ref[...], preferred_element_type=jnp.float32,
                precision=_PREC) + b2_ref[...]
    h = _layernorm(h, g2_ref[...], beta2_ref[...])
    h = _gelu_exact(h)
    logits = jnp.dot(h, w3_ref[...], preferred_element_type=jnp.float32,
                     precision=_PREC)
    l_out_ref[...] = logits

    lane = jax.lax.broadcasted_iota(jnp.int32, (BT, NUM_EXPERTS), 1)
    work = logits
    vals, idxs = [], []
    for _ in range(TOP_K):
        m = jnp.max(work, axis=-1, keepdims=True)
        is_m = work == m
        idx = jnp.min(jnp.where(is_m, lane, NUM_EXPERTS), axis=-1,
                      keepdims=True)
        vals.append(m)
        idxs.append(idx)
        work = jnp.where(lane == idx, -jnp.inf, work)
    v = jnp.concatenate(vals, axis=-1)
    i = jnp.concatenate(idxs, axis=-1)
    e = jnp.exp(v - v[:, 0:1])
    w_out_ref[...] = e / jnp.sum(e, axis=-1, keepdims=True)
    i_out_ref[...] = i


@jax.jit
def kernel(x, W1, b1, g1, beta1, W2, b2, g2, beta2, W3):
    grid = (N_TOKENS // BT,)
    full = lambda shape: pl.BlockSpec(shape, lambda i: (0,) * len(shape))
    out_shapes = (
        jax.ShapeDtypeStruct((N_TOKENS, TOP_K), jnp.float32),
        jax.ShapeDtypeStruct((N_TOKENS, TOP_K), jnp.int32),
        jax.ShapeDtypeStruct((N_TOKENS, NUM_EXPERTS), jnp.float32),
    )
    return pl.pallas_call(
        _gate_body,
        grid=grid,
        in_specs=[
            pl.BlockSpec((BT, MODEL_DIM), lambda i: (i, 0)),
            full((MODEL_DIM, H1)),
            full((H1,)),
            full((H1,)),
            full((H1,)),
            full((H1, H2)),
            full((H2,)),
            full((H2,)),
            full((H2,)),
            full((H2, NUM_EXPERTS)),
        ],
        out_specs=(
            pl.BlockSpec((BT, TOP_K), lambda i: (i, 0)),
            pl.BlockSpec((BT, TOP_K), lambda i: (i, 0)),
            pl.BlockSpec((BT, NUM_EXPERTS), lambda i: (i, 0)),
        ),
        out_shape=out_shapes,
    )(x, W1, b1, g1, beta1, W2, b2, g2, beta2, W3)
